# baseline (speedup 1.0000x reference)
import functools

import jax
import jax.numpy as jnp
from jax.experimental import pallas as pl
from jax.experimental.pallas import tpu as pltpu

_EPS = 1e-6
_LANES = 128


def _gem_kernel(x_ref, pbc_ref, o_ref, *, hw, n_total):
    n = pl.program_id(0)
    pbc = pbc_ref[...]
    xm = jnp.maximum(x_ref[...], _EPS)
    xp = jnp.exp2(jnp.log2(xm) * pbc)
    s_col = jnp.sum(xp, axis=-1, keepdims=True)
    lane = jax.lax.broadcasted_iota(jnp.int32, o_ref.shape, 1)
    o_ref[...] = jnp.where(lane == n, s_col, o_ref[...])

    @pl.when(n == n_total - 1)
    def _():
        s = o_ref[...]
        pbn = pbc[:, :s.shape[1]]
        pad_terms = (_LANES - hw) * jnp.exp2(jnp.log2(_EPS) * pbn)
        m = (s - pad_terms) * (1.0 / hw)
        o_ref[...] = jnp.exp2(jnp.log2(m) * (1.0 / pbn))


def kernel(x, p):
    N, C, H, W = x.shape
    HW = H * W

    xp4 = jnp.pad(x.astype(jnp.float32),
                  ((0, 0), (0, 0), (0, 8 - H), (0, 16 - W)),
                  constant_values=0.0)
    x_pad = xp4.reshape(N * C, _LANES)
    p_bc = jnp.broadcast_to(p.astype(jnp.float32).reshape(C, 1),
                            (C, _LANES))

    out_t = pl.pallas_call(
        functools.partial(_gem_kernel, hw=HW, n_total=N),
        out_shape=jax.ShapeDtypeStruct((C, N), jnp.float32),
        grid=(N,),
        in_specs=[
            pl.BlockSpec((C, _LANES), lambda n: (n, 0)),
            pl.BlockSpec((C, _LANES), lambda n: (0, 0)),
        ],
        out_specs=pl.BlockSpec((C, N), lambda n: (0, 0)),
        compiler_params=pltpu.CompilerParams(
            dimension_semantics=("arbitrary",)),
    )(x_pad, p_bc)

    return out_t.T.reshape(N, C, 1, 1)


# speedup vs baseline: 1.3497x; 1.3497x over previous
import functools

import jax
import jax.numpy as jnp
from jax.experimental import pallas as pl
from jax.experimental.pallas import tpu as pltpu

_EPS = 1e-6
_LANES = 128


def _gem_kernel(x_ref, pbc_ref, o_ref, *, hw, n_total):
    n = pl.program_id(0)
    pbc = pbc_ref[...]
    xm = jnp.maximum(x_ref[...], _EPS)
    xp = jnp.exp2(jnp.log2(xm) * pbc[:, :hw])
    s_col = jnp.sum(xp, axis=-1, keepdims=True)
    lane = jax.lax.broadcasted_iota(jnp.int32, o_ref.shape, 1)
    o_ref[...] = jnp.where(lane == n, s_col, o_ref[...])

    @pl.when(n == n_total - 1)
    def _():
        s = o_ref[...]
        pbn = pbc[:, :s.shape[1]]
        m = s * (1.0 / hw)
        o_ref[...] = jnp.exp2(jnp.log2(m) * (1.0 / pbn))


def kernel(x, p):
    N, C, H, W = x.shape
    HW = H * W

    x2 = x.astype(jnp.float32).reshape(N * C, HW)
    p_bc = jnp.broadcast_to(p.astype(jnp.float32).reshape(C, 1),
                            (C, _LANES))

    out_t = pl.pallas_call(
        functools.partial(_gem_kernel, hw=HW, n_total=N),
        out_shape=jax.ShapeDtypeStruct((C, N), jnp.float32),
        grid=(N,),
        in_specs=[
            pl.BlockSpec((C, HW), lambda n: (n, 0)),
            pl.BlockSpec((C, _LANES), lambda n: (0, 0)),
        ],
        out_specs=pl.BlockSpec((C, N), lambda n: (0, 0)),
        compiler_params=pltpu.CompilerParams(
            dimension_semantics=("arbitrary",)),
    )(x2, p_bc)

    return out_t.T.reshape(N, C, 1, 1)


# speedup vs baseline: 1.3533x; 1.0026x over previous
import functools

import jax
import jax.numpy as jnp
from jax.experimental import pallas as pl
from jax.experimental.pallas import tpu as pltpu

_EPS = 1e-6
_LANES = 128


def _gem_kernel(x_ref, pbc_ref, o_ref, acc_ref, *, hw, n_total):
    n = pl.program_id(0)
    pbc = pbc_ref[...]
    xm = jnp.maximum(x_ref[...], _EPS)
    xp = jnp.exp2(jnp.log2(xm) * pbc[:, :hw])
    s_col = jnp.sum(xp, axis=-1, keepdims=True)
    lane = jax.lax.broadcasted_iota(jnp.int32, acc_ref.shape, 1)
    acc_ref[...] = jnp.where(lane == n, s_col, acc_ref[...])

    @pl.when(n == n_total - 1)
    def _():
        s = acc_ref[...]
        pbn = pbc[:, :s.shape[1]]
        m = s * (1.0 / hw)
        y = jnp.exp2(jnp.log2(m) * (1.0 / pbn))
        o_ref[...] = y.T


def kernel(x, p):
    N, C, H, W = x.shape
    HW = H * W

    x2 = x.astype(jnp.float32).reshape(N * C, HW)
    p_bc = jnp.broadcast_to(p.astype(jnp.float32).reshape(C, 1),
                            (C, _LANES))

    out = pl.pallas_call(
        functools.partial(_gem_kernel, hw=HW, n_total=N),
        out_shape=jax.ShapeDtypeStruct((N, C), jnp.float32),
        grid=(N,),
        in_specs=[
            pl.BlockSpec((C, HW), lambda n: (n, 0)),
            pl.BlockSpec((C, _LANES), lambda n: (0, 0)),
        ],
        out_specs=pl.BlockSpec((N, C), lambda n: (0, 0)),
        scratch_shapes=[pltpu.VMEM((C, N), jnp.float32)],
        compiler_params=pltpu.CompilerParams(
            dimension_semantics=("arbitrary",)),
    )(x2, p_bc)

    return out.reshape(N, C, 1, 1)


# speedup vs baseline: 1.9672x; 1.4536x over previous
import functools

import jax
import jax.numpy as jnp
from jax.experimental import pallas as pl
from jax.experimental.pallas import tpu as pltpu

_EPS = 1e-6
_LANES = 128


def _gem_kernel(x_ref, pbc_ref, o_ref, acc_ref, *, hw, n_total):
    n = pl.program_id(0)
    pbc = pbc_ref[...]
    xm = jnp.maximum(x_ref[0], _EPS)
    xp = jnp.exp2(jnp.log2(xm) * pbc[:, :hw])
    s_col = jnp.sum(xp, axis=-1, keepdims=True)
    lane = jax.lax.broadcasted_iota(jnp.int32, acc_ref.shape, 1)
    acc_ref[...] = jnp.where(lane == n, s_col, acc_ref[...])

    @pl.when(n == n_total - 1)
    def _():
        s = acc_ref[...]
        pbn = pbc[:, :s.shape[1]]
        m = s * (1.0 / hw)
        y = jnp.exp2(jnp.log2(m) * (1.0 / pbn))
        o_ref[...] = y.T


def kernel(x, p):
    N, C, H, W = x.shape
    HW = H * W

    x2 = x.reshape(N, C, HW)
    p_bc = jnp.broadcast_to(p.astype(jnp.float32).reshape(C, 1),
                            (C, _LANES))

    out = pl.pallas_call(
        functools.partial(_gem_kernel, hw=HW, n_total=N),
        out_shape=jax.ShapeDtypeStruct((N, C), jnp.float32),
        grid=(N,),
        in_specs=[
            pl.BlockSpec((1, C, HW), lambda n: (n, 0, 0)),
            pl.BlockSpec((C, _LANES), lambda n: (0, 0)),
        ],
        out_specs=pl.BlockSpec((N, C), lambda n: (0, 0)),
        scratch_shapes=[pltpu.VMEM((C, N), jnp.float32)],
        compiler_params=pltpu.CompilerParams(
            dimension_semantics=("arbitrary",)),
    )(x2, p_bc)

    return out.reshape(N, C, 1, 1)


# speedup vs baseline: 2.6391x; 1.3416x over previous
import functools

import jax
import jax.numpy as jnp
from jax.experimental import pallas as pl
from jax.experimental.pallas import tpu as pltpu

_EPS = 1e-6
_LANES = 128


def _gem_kernel(x_ref, pbc_ref, o_ref, acc_ref, *, hw, g, n_steps):
    n = pl.program_id(0)
    pbc = pbc_ref[...]
    xm = jnp.maximum(x_ref[...], _EPS)
    xp = jnp.exp2(jnp.log2(xm) * pbc[None, :, :hw])
    s3 = jnp.sum(xp, axis=-1, keepdims=True)
    lane = jax.lax.broadcasted_iota(jnp.int32, acc_ref.shape, 1)
    acc = acc_ref[...]
    for i in range(g):
        acc = jnp.where(lane == n * g + i, s3[i], acc)
    acc_ref[...] = acc

    @pl.when(n == n_steps - 1)
    def _():
        s = acc_ref[...]
        pbn = pbc[:, :s.shape[1]]
        m = s * (1.0 / hw)
        y = jnp.exp2(jnp.log2(m) * (1.0 / pbn))
        o_ref[...] = y.T


def kernel(x, p):
    N, C, H, W = x.shape
    HW = H * W

    x2 = x.reshape(N, C, HW)
    p_bc = jnp.broadcast_to(p.astype(jnp.float32).reshape(C, 1),
                            (C, _LANES))

    g = 4
    while g > 1 and N % g:
        g //= 2
    out = pl.pallas_call(
        functools.partial(_gem_kernel, hw=HW, g=g, n_steps=N // g),
        out_shape=jax.ShapeDtypeStruct((N, C), jnp.float32),
        grid=(N // g,),
        in_specs=[
            pl.BlockSpec((g, C, HW), lambda n: (n, 0, 0)),
            pl.BlockSpec((C, _LANES), lambda n: (0, 0)),
        ],
        out_specs=pl.BlockSpec((N, C), lambda n: (0, 0)),
        scratch_shapes=[pltpu.VMEM((C, N), jnp.float32)],
        compiler_params=pltpu.CompilerParams(
            dimension_semantics=("arbitrary",)),
    )(x2, p_bc)

    return out.reshape(N, C, 1, 1)


# speedup vs baseline: 2.7946x; 1.0589x over previous
import functools

import jax
import jax.numpy as jnp
from jax.experimental import pallas as pl
from jax.experimental.pallas import tpu as pltpu

_EPS = 1e-6
_LANES = 128


def _gem_kernel(x_ref, pbc_ref, o_ref, acc_ref, *, hw, g, n_steps):
    n = pl.program_id(0)
    pbc = pbc_ref[...]
    xm = jnp.maximum(x_ref[...], _EPS)
    xp = jnp.exp2(jnp.log2(xm) * pbc[None, :, :hw])
    s3 = jnp.sum(xp, axis=-1, keepdims=True)
    lane = jax.lax.broadcasted_iota(jnp.int32, acc_ref.shape, 1)
    acc = acc_ref[...]
    for i in range(g):
        acc = jnp.where(lane == n * g + i, s3[i], acc)
    acc_ref[...] = acc

    @pl.when(n == n_steps - 1)
    def _():
        s = acc_ref[...]
        pbn = pbc[:, :s.shape[1]]
        m = s * (1.0 / hw)
        y = jnp.exp2(jnp.log2(m) * (1.0 / pbn))
        o_ref[...] = y.T


def kernel(x, p):
    N, C, H, W = x.shape
    HW = H * W

    x2 = x.reshape(N, C, HW)
    p_bc = jnp.broadcast_to(p.astype(jnp.float32).reshape(C, 1),
                            (C, _LANES))

    g = 8
    while g > 1 and N % g:
        g //= 2
    out = pl.pallas_call(
        functools.partial(_gem_kernel, hw=HW, g=g, n_steps=N // g),
        out_shape=jax.ShapeDtypeStruct((N, C), jnp.float32),
        grid=(N // g,),
        in_specs=[
            pl.BlockSpec((g, C, HW), lambda n: (n, 0, 0)),
            pl.BlockSpec((C, _LANES), lambda n: (0, 0)),
        ],
        out_specs=pl.BlockSpec((N, C), lambda n: (0, 0)),
        scratch_shapes=[pltpu.VMEM((C, N), jnp.float32)],
        compiler_params=pltpu.CompilerParams(
            dimension_semantics=("arbitrary",)),
    )(x2, p_bc)

    return out.reshape(N, C, 1, 1)


# speedup vs baseline: 2.8193x; 1.0088x over previous
import functools

import jax
import jax.numpy as jnp
from jax.experimental import pallas as pl
from jax.experimental.pallas import tpu as pltpu

_EPS = 1e-6
_LANES = 128


def _gem_kernel(x_ref, pbc_ref, o_ref, acc_ref, *, hw, g, n_steps):
    n = pl.program_id(0)
    pbc = pbc_ref[...]
    xm = jnp.maximum(x_ref[...], _EPS)
    xp = jnp.exp2(jnp.log2(xm) * pbc[None, :, :hw])
    s3 = jnp.sum(xp, axis=-1, keepdims=True)
    lane = jax.lax.broadcasted_iota(jnp.int32, acc_ref.shape, 1)
    acc = acc_ref[...]
    for i in range(g):
        acc = jnp.where(lane == n * g + i, s3[i], acc)
    acc_ref[...] = acc

    @pl.when(n == n_steps - 1)
    def _():
        s = acc_ref[...]
        pbn = pbc[:, :s.shape[1]]
        m = s * (1.0 / hw)
        y = jnp.exp2(jnp.log2(m) * (1.0 / pbn))
        o_ref[...] = y.T


def kernel(x, p):
    N, C, H, W = x.shape
    HW = H * W

    x2 = x.reshape(N, C, HW)
    p_bc = jnp.broadcast_to(p.astype(jnp.float32).reshape(C, 1),
                            (C, _LANES))

    g = 16
    while g > 1 and N % g:
        g //= 2
    out = pl.pallas_call(
        functools.partial(_gem_kernel, hw=HW, g=g, n_steps=N // g),
        out_shape=jax.ShapeDtypeStruct((N, C), jnp.float32),
        grid=(N // g,),
        in_specs=[
            pl.BlockSpec((g, C, HW), lambda n: (n, 0, 0)),
            pl.BlockSpec((C, _LANES), lambda n: (0, 0)),
        ],
        out_specs=pl.BlockSpec((N, C), lambda n: (0, 0)),
        scratch_shapes=[pltpu.VMEM((C, N), jnp.float32)],
        compiler_params=pltpu.CompilerParams(
            dimension_semantics=("arbitrary",)),
    )(x2, p_bc)

    return out.reshape(N, C, 1, 1)
